# baseline (speedup 1.0000x reference)
import functools

import jax
import jax.numpy as jnp
from jax import lax
from jax.experimental import pallas as pl
from jax.experimental.pallas import tpu as pltpu
from jax.experimental.pallas import tpu_sc as plsc

_NUM_CORES = 2
_NUM_SUBCORES = 16
_NUM_WORKERS = _NUM_CORES * _NUM_SUBCORES
_LANES = 16


def _sc_body(chunk, d, table_hbm, idxh_hbm, par_hbm, o0_hbm, o1_hbm,
             idxh_v, par_v, rows_v, buf0, buf1, sem_g, sem0, sem1):
    wid = lax.axis_index("s") * _NUM_CORES + lax.axis_index("c")
    base = wid * chunk

    pltpu.sync_copy(idxh_hbm, idxh_v)
    pltpu.sync_copy(par_hbm, par_v)
    pltpu.async_copy(table_hbm.at[idxh_v], rows_v, sem_g).wait()

    nvec = d // _LANES
    p0 = par_v[0, :] > 0
    p1 = par_v[1, :] > 0
    row0 = [
        jnp.where(p0, rows_v[0, pl.ds(d + j * _LANES, _LANES)],
                  rows_v[0, pl.ds(j * _LANES, _LANES)])
        for j in range(nvec)
    ]
    row1 = [
        jnp.where(p1, rows_v[1, pl.ds(d + j * _LANES, _LANES)],
                  rows_v[1, pl.ds(j * _LANES, _LANES)])
        for j in range(nvec)
    ]

    rep = buf0.shape[0]

    def fill_row(r, carry):
        for j in range(nvec):
            buf0[r, pl.ds(j * _LANES, _LANES)] = row0[j]
            buf1[r, pl.ds(j * _LANES, _LANES)] = row1[j]
        return carry

    lax.fori_loop(0, rep, fill_row, 0)

    cps = []
    for t in range(chunk // rep):
        cps.append(pltpu.make_async_copy(
            buf0, o0_hbm.at[pl.ds(base + t * rep, rep)], sem0))
        cps.append(pltpu.make_async_copy(
            buf1, o1_hbm.at[pl.ds(base + t * rep, rep)], sem1))
    for cp in cps:
        cp.start()
    for cp in cps:
        cp.wait()


def kernel(rel, table, indices):
    batch = rel.shape[0]
    vocab, d = table.shape
    chunk = batch // _NUM_WORKERS
    idx = indices.astype(jnp.int32)
    idxh = idx // 2
    par = jnp.broadcast_to((idx % 2)[:, None], (2, _LANES))
    table2 = table.reshape(vocab // 2, 2 * d)

    mesh = plsc.VectorSubcoreMesh(core_axis_name="c", subcore_axis_name="s")
    out_sds = jax.ShapeDtypeStruct((batch, d), jnp.float32)
    sc_call = pl.kernel(
        functools.partial(_sc_body, chunk, d),
        out_type=[out_sds, out_sds],
        mesh=mesh,
        scratch_types=[
            pltpu.VMEM((2,), jnp.int32),
            pltpu.VMEM((2, _LANES), jnp.int32),
            pltpu.VMEM((2, 2 * d), jnp.float32),
            pltpu.VMEM((chunk // 2, d), jnp.float32),
            pltpu.VMEM((chunk // 2, d), jnp.float32),
            pltpu.SemaphoreType.DMA,
            pltpu.SemaphoreType.DMA,
            pltpu.SemaphoreType.DMA,
        ],
    )
    o0, o1 = sc_call(table2, idxh, par)
    return (o0, o1)


# speedup vs baseline: 1.3630x; 1.3630x over previous
import functools

import jax
import jax.numpy as jnp
from jax import lax
from jax.experimental import pallas as pl
from jax.experimental.pallas import tpu as pltpu
from jax.experimental.pallas import tpu_sc as plsc

_NUM_CORES = 2
_NUM_SUBCORES = 16
_NUM_WORKERS = _NUM_CORES * _NUM_SUBCORES
_LANES = 16


def _tc_gather_body(idx_ref, table_ref, rows_ref, sem):
    for k in range(2):
        cp = pltpu.make_async_copy(
            table_ref.at[pl.ds(idx_ref[k], 1)], rows_ref.at[pl.ds(k, 1)], sem)
        cp.start()
        cp.wait()


def _sc_body(chunk, d, rows_hbm, o0_hbm, o1_hbm,
             rows_v, buf0, buf1, sem0, sem1):
    wid = lax.axis_index("s") * _NUM_CORES + lax.axis_index("c")
    base = wid * chunk

    pltpu.sync_copy(rows_hbm, rows_v)

    nvec = d // _LANES
    row0 = [rows_v[0, pl.ds(j * _LANES, _LANES)] for j in range(nvec)]
    row1 = [rows_v[1, pl.ds(j * _LANES, _LANES)] for j in range(nvec)]

    rep = buf0.shape[0]

    def fill_row(r, carry):
        for j in range(nvec):
            buf0[r, pl.ds(j * _LANES, _LANES)] = row0[j]
            buf1[r, pl.ds(j * _LANES, _LANES)] = row1[j]
        return carry

    lax.fori_loop(0, rep, fill_row, 0)

    cps = []
    for t in range(chunk // rep):
        cps.append(pltpu.make_async_copy(
            buf0, o0_hbm.at[pl.ds(base + t * rep, rep)], sem0))
        cps.append(pltpu.make_async_copy(
            buf1, o1_hbm.at[pl.ds(base + t * rep, rep)], sem1))
    for cp in cps:
        cp.start()
    for cp in cps:
        cp.wait()


def kernel(rel, table, indices):
    batch = rel.shape[0]
    d = table.shape[1]
    chunk = batch // _NUM_WORKERS

    rows = pl.pallas_call(
        _tc_gather_body,
        grid_spec=pltpu.PrefetchScalarGridSpec(
            num_scalar_prefetch=1,
            grid=(1,),
            in_specs=[pl.BlockSpec(memory_space=pl.ANY)],
            out_specs=pl.BlockSpec((2, d), lambda i, idx: (0, 0)),
            scratch_shapes=[pltpu.SemaphoreType.DMA],
        ),
        out_shape=jax.ShapeDtypeStruct((2, d), jnp.float32),
    )(indices.astype(jnp.int32), table)

    mesh = plsc.VectorSubcoreMesh(core_axis_name="c", subcore_axis_name="s")
    out_sds = jax.ShapeDtypeStruct((batch, d), jnp.float32)
    sc_call = pl.kernel(
        functools.partial(_sc_body, chunk, d),
        out_type=[out_sds, out_sds],
        mesh=mesh,
        scratch_types=[
            pltpu.VMEM((2, d), jnp.float32),
            pltpu.VMEM((chunk // 2, d), jnp.float32),
            pltpu.VMEM((chunk // 2, d), jnp.float32),
            pltpu.SemaphoreType.DMA,
            pltpu.SemaphoreType.DMA,
        ],
    )
    o0, o1 = sc_call(rows)
    return (o0, o1)


# speedup vs baseline: 13.6713x; 10.0307x over previous
import jax
import jax.numpy as jnp
from jax.experimental import pallas as pl
from jax.experimental.pallas import tpu as pltpu

_BLOCK_B = 2048


def _tc_body(idx_ref, table_ref, o0_ref, o1_ref, cols, win, sem):
    i = pl.program_id(0)

    @pl.when(i == 0)
    def _fetch_cols():
        d = win.shape[0]
        for k in range(2):
            base = (idx_ref[k] // 128) * 128
            cp = pltpu.make_async_copy(
                table_ref.at[:, pl.ds(base, 128)], win, sem)
            cp.start()
            cp.wait()
            off = idx_ref[k] % 128
            lane = jax.lax.broadcasted_iota(jnp.int32, (d, 128), 1)
            colk = jnp.sum(
                jnp.where(lane == off, win[...], 0.0), axis=1, keepdims=True)
            cols[:, pl.ds(k, 1)] = colk

    o0_ref[...] = jnp.broadcast_to(cols[:, 0:1], o0_ref.shape)
    o1_ref[...] = jnp.broadcast_to(cols[:, 1:2], o1_ref.shape)


def kernel(rel, table, indices):
    batch = rel.shape[0]
    d = table.shape[1]
    table_t = table.T
    grid = (batch // _BLOCK_B,)
    out_sds = jax.ShapeDtypeStruct((d, batch), jnp.float32)
    o0, o1 = pl.pallas_call(
        _tc_body,
        grid_spec=pltpu.PrefetchScalarGridSpec(
            num_scalar_prefetch=1,
            grid=grid,
            in_specs=[pl.BlockSpec(memory_space=pl.ANY)],
            out_specs=[
                pl.BlockSpec((d, _BLOCK_B), lambda i, idx: (0, i)),
                pl.BlockSpec((d, _BLOCK_B), lambda i, idx: (0, i)),
            ],
            scratch_shapes=[
                pltpu.VMEM((d, 2), jnp.float32),
                pltpu.VMEM((d, 128), jnp.float32),
                pltpu.SemaphoreType.DMA,
            ],
        ),
        out_shape=[out_sds, out_sds],
    )(indices.astype(jnp.int32), table_t)
    return (o0.T, o1.T)
